# baseline (speedup 1.0000x reference)
import functools

import jax
import jax.numpy as jnp
from jax import lax
from jax.experimental import pallas as pl
from jax.experimental.pallas import tpu as pltpu
from jax.experimental.pallas import tpu_sc as plsc

H = 128
NB = 10
NA = 10000
NBOND = 20000
NMESS = 10000
NTAB = NMESS + NBOND
AF = 35
BF = 40
NMOL = 200
APM = 50

NC = 2
NS = 16
NW = NC * NS
CHUNK = 8
LG = H // 16


_RING = 8


def _make_gather_sum(n_pad, r0, r1):
    assert NS * (r0 + r1) == n_pad
    nch0, nch1 = r0 // CHUNK, r1 // CHUNK
    assert nch0 % _RING == 0 and nch1 % _RING == 0
    nch_max = max(nch0, nch1)
    mesh = plsc.VectorSubcoreMesh(
        core_axis_name="c", subcore_axis_name="s", num_cores=NC, num_subcores=NS
    )

    @functools.partial(
        pl.kernel,
        out_type=jax.ShapeDtypeStruct((n_pad, H), jnp.float32),
        mesh=mesh,
        scratch_types=[
            pltpu.VMEM((nch_max, 80), jnp.int32),
            pltpu.VMEM((_RING, 80, H), jnp.float32),
            pltpu.VMEM((2, CHUNK, H), jnp.float32),
            [pltpu.SemaphoreType.DMA] * _RING,
            [pltpu.SemaphoreType.DMA] * 2,
        ],
    )
    def gather_sum(table_hbm, idx_hbm, out_hbm, idx_v, rows_v, out_v, gsems, ssems):
        cid = lax.axis_index("c")
        sid = lax.axis_index("s")
        base = pl.multiple_of(
            jnp.where(cid == 0, sid * r0, NS * r0 + sid * r1), 64)
        n_chunks = jnp.where(cid == 0, nch0, nch1)
        cbase = pl.multiple_of(base // CHUNK, 8)

        @pl.when(cid == 0)
        def _():
            pltpu.sync_copy(idx_hbm.at[pl.ds(cbase, nch0)],
                            idx_v.at[pl.ds(0, nch0)])

        @pl.when(cid == 1)
        def _():
            pltpu.sync_copy(idx_hbm.at[pl.ds(cbase, nch1)],
                            idx_v.at[pl.ds(0, nch1)])

        def issue(c, slot):
            pltpu.async_copy(
                table_hbm.at[idx_v.at[c]], rows_v.at[slot], gsems[slot],
            )

        def wait_gather(c, slot):
            pltpu.make_async_copy(
                table_hbm.at[idx_v.at[c]], rows_v.at[slot], gsems[slot],
            ).wait()

        def wait_store(c, b):
            pltpu.make_async_copy(
                out_v.at[b], out_hbm.at[pl.ds(base + CHUNK * c, CHUNK)], ssems[b]
            ).wait()

        for s in range(_RING - 1):
            issue(s, s)

        @pl.loop(0, (nch_max + _RING - 1) // _RING)
        def _chunk(c4):
            for s in range(_RING):
                c = _RING * c4 + s
                @pl.when(c < n_chunks)
                def _():
                    nxt = (s + _RING - 1) % _RING
                    @pl.when(c + _RING - 1 < n_chunks)
                    def _():
                        issue(c + _RING - 1, nxt)
                    wait_gather(c, s)
                    b = s % 2
                    @pl.when(c >= 2)
                    def _():
                        wait_store(c - 2, b)
                    @pl.loop(0, CHUNK)
                    def _sum(t):
                        for h in range(LG):
                            acc = rows_v[s, NB * t, pl.ds(16 * h, 16)]
                            for r in range(1, NB):
                                acc = acc + rows_v[s, NB * t + r,
                                                   pl.ds(16 * h, 16)]
                            out_v[b, t, pl.ds(16 * h, 16)] = acc
                    pltpu.async_copy(
                        out_v.at[b], out_hbm.at[pl.ds(base + CHUNK * c, CHUNK)],
                        ssems[b]
                    )

        wait_store(n_chunks - 2, 0)
        wait_store(n_chunks - 1, 1)

    return gather_sum


_PAD_B = 20480
_PAD_A = 10240
_R0_B, _R1_B = 832, 448
_R0_A, _R1_A = 448, 192
_gather_sum_b = _make_gather_sum(_PAD_B, _R0_B, _R1_B)
_gather_sum_a = _make_gather_sum(_PAD_A, _R0_A, _R1_A)


def _prep_idx(g, n_pad):
    g = g.astype(jnp.int32)
    n = g.shape[0]
    g = jnp.concatenate([g, jnp.zeros((n_pad - n, NB), jnp.int32)], axis=0)
    return g.reshape(n_pad // CHUNK, 80)


_BB = 1000
_NTREE_BLK = NMESS // _BB
_NTAB_BLK = NTAB // _BB


def _binput_body(fb_ref, wi_ref, out_ref):
    out_ref[...] = jnp.dot(fb_ref[...], wi_ref[...],
                           preferred_element_type=jnp.float32)


def _tc_binput(fbonds, W_i):
    return pl.pallas_call(
        _binput_body,
        grid=(NBOND // _BB,),
        in_specs=[
            pl.BlockSpec((_BB, BF), lambda i: (i, 0)),
            pl.BlockSpec((BF, H), lambda i: (0, 0)),
        ],
        out_specs=pl.BlockSpec((_BB, H), lambda i: (i, 0)),
        out_shape=jax.ShapeDtypeStruct((NBOND, H), jnp.float32),
    )(fbonds, W_i)


def _table0_body(tree_ref, bin_ref, out_ref):
    i = pl.program_id(0)

    @pl.when(i < _NTREE_BLK)
    def _():
        out_ref[...] = tree_ref[...]

    @pl.when(i >= _NTREE_BLK)
    def _():
        out_ref[...] = jnp.maximum(bin_ref[...], 0.0)


def _tc_table0(tree, binput):
    return pl.pallas_call(
        _table0_body,
        grid=(_NTAB_BLK,),
        in_specs=[
            pl.BlockSpec((_BB, H), lambda i: (jnp.minimum(i, _NTREE_BLK - 1), 0)),
            pl.BlockSpec((_BB, H), lambda i: (jnp.maximum(i - _NTREE_BLK, 0), 0)),
        ],
        out_specs=pl.BlockSpec((_BB, H), lambda i: (i, 0)),
        out_shape=jax.ShapeDtypeStruct((NTAB, H), jnp.float32),
    )(tree, binput)


def _iter_body(tab_ref, bin_ref, nei_ref, wh_ref, out_ref):
    del tab_ref
    acc = jnp.dot(nei_ref[...], wh_ref[...], preferred_element_type=jnp.float32)
    out_ref[...] = jnp.maximum(bin_ref[...] + acc, 0.0)


def _tc_iter(table, binput, nei, W_h):
    return pl.pallas_call(
        _iter_body,
        grid=(NBOND // _BB,),
        in_specs=[
            pl.BlockSpec(memory_space=pl.ANY),
            pl.BlockSpec((_BB, H), lambda i: (i, 0)),
            pl.BlockSpec((_BB, H), lambda i: (i, 0)),
            pl.BlockSpec((H, H), lambda i: (0, 0)),
        ],
        out_specs=pl.BlockSpec((_BB, H), lambda i: (i + _NTREE_BLK, 0)),
        out_shape=jax.ShapeDtypeStruct((NTAB, H), jnp.float32),
        input_output_aliases={0: 0},
    )(table, binput, nei, W_h)


_AB = 2000
_MPB = _AB // APM


def _out_body(fa_ref, nei_ref, woa_ref, won_ref, bo_ref, out_ref):
    hdn = jnp.dot(fa_ref[...], woa_ref[...], preferred_element_type=jnp.float32)
    hdn += jnp.dot(nei_ref[...], won_ref[...], preferred_element_type=jnp.float32)
    hdn = jnp.maximum(hdn + bo_ref[...], 0.0)
    rows = lax.broadcasted_iota(jnp.int32, (_MPB, _AB), 1) // APM
    mols = lax.broadcasted_iota(jnp.int32, (_MPB, _AB), 0)
    pool = jnp.where(rows == mols, 1.0 / APM, 0.0)
    out_ref[...] = jnp.dot(pool, hdn, preferred_element_type=jnp.float32)


def _tc_out(fatoms, nei_a, W_oa, W_on, b_o):
    return pl.pallas_call(
        _out_body,
        grid=(NA // _AB,),
        in_specs=[
            pl.BlockSpec((_AB, AF), lambda i: (i, 0)),
            pl.BlockSpec((_AB, H), lambda i: (i, 0)),
            pl.BlockSpec((AF, H), lambda i: (0, 0)),
            pl.BlockSpec((H, H), lambda i: (0, 0)),
            pl.BlockSpec((1, H), lambda i: (0, 0)),
        ],
        out_specs=pl.BlockSpec((_MPB, H), lambda i: (i, 0)),
        out_shape=jax.ShapeDtypeStruct((NMOL, H), jnp.float32),
    )(fatoms, nei_a, W_oa, W_on, b_o)


def kernel(fatoms, fbonds, tree_message, W_i, W_h, W_o, b_o, agraph, bgraph):
    idx_b = _prep_idx(bgraph, _PAD_B)
    idx_a = _prep_idx(agraph, _PAD_A)

    binput = _tc_binput(fbonds, W_i)
    table = _tc_table0(tree_message, binput)
    for _ in range(3):
        nei = _gather_sum_b(table, idx_b)[:NBOND]
        table = _tc_iter(table, binput, nei, W_h)
    nei_a = _gather_sum_a(table, idx_a)[:NA]
    return _tc_out(fatoms, nei_a, W_o[:AF], W_o[AF:], b_o.reshape(1, H))


# speedup vs baseline: 1.0159x; 1.0159x over previous
import functools

import jax
import jax.numpy as jnp
from jax import lax
from jax.experimental import pallas as pl
from jax.experimental.pallas import tpu as pltpu
from jax.experimental.pallas import tpu_sc as plsc

H = 128
NB = 10
NA = 10000
NBOND = 20000
NMESS = 10000
NTAB = NMESS + NBOND
AF = 35
BF = 40
NMOL = 200
APM = 50

NC = 2
NS = 16
NW = NC * NS
CHUNK = 8
LG = H // 16


_RING = 8


def _make_gather_sum(n_pad, r0, r1):
    assert NS * (r0 + r1) == n_pad
    nch0, nch1 = r0 // CHUNK, r1 // CHUNK
    assert nch0 % _RING == 0 and nch1 % _RING == 0
    nch_max = max(nch0, nch1)
    mesh = plsc.VectorSubcoreMesh(
        core_axis_name="c", subcore_axis_name="s", num_cores=NC, num_subcores=NS
    )

    @functools.partial(
        pl.kernel,
        out_type=jax.ShapeDtypeStruct((n_pad, H), jnp.float32),
        mesh=mesh,
        scratch_types=[
            pltpu.VMEM((nch_max, 80), jnp.int32),
            pltpu.VMEM((_RING, 80, H), jnp.float32),
            pltpu.VMEM((2, CHUNK, H), jnp.float32),
            [pltpu.SemaphoreType.DMA] * _RING,
            [pltpu.SemaphoreType.DMA] * 2,
        ],
    )
    def gather_sum(table_hbm, idx_hbm, out_hbm, idx_v, rows_v, out_v, gsems, ssems):
        cid = lax.axis_index("c")
        sid = lax.axis_index("s")
        base = pl.multiple_of(
            jnp.where(cid == 0, sid * r0, NS * r0 + sid * r1), 64)
        n_chunks = jnp.where(cid == 0, nch0, nch1)
        cbase = pl.multiple_of(base // CHUNK, 8)

        @pl.when(cid == 0)
        def _():
            pltpu.sync_copy(idx_hbm.at[pl.ds(cbase, nch0)],
                            idx_v.at[pl.ds(0, nch0)])

        @pl.when(cid == 1)
        def _():
            pltpu.sync_copy(idx_hbm.at[pl.ds(cbase, nch1)],
                            idx_v.at[pl.ds(0, nch1)])

        def issue(c, slot):
            pltpu.async_copy(
                table_hbm.at[idx_v.at[c]], rows_v.at[slot], gsems[slot],
            )

        def wait_gather(c, slot):
            pltpu.make_async_copy(
                table_hbm.at[idx_v.at[c]], rows_v.at[slot], gsems[slot],
            ).wait()

        def wait_store(c, b):
            pltpu.make_async_copy(
                out_v.at[b], out_hbm.at[pl.ds(base + CHUNK * c, CHUNK)], ssems[b]
            ).wait()

        for s in range(_RING - 1):
            issue(s, s)

        @pl.loop(0, (nch_max + _RING - 1) // _RING)
        def _chunk(c4):
            for s in range(_RING):
                c = _RING * c4 + s
                @pl.when(c < n_chunks)
                def _():
                    nxt = (s + _RING - 1) % _RING
                    @pl.when(c + _RING - 1 < n_chunks)
                    def _():
                        issue(c + _RING - 1, nxt)
                    wait_gather(c, s)
                    b = s % 2
                    @pl.when(c >= 2)
                    def _():
                        wait_store(c - 2, b)
                    @pl.loop(0, CHUNK)
                    def _sum(t):
                        for h in range(LG):
                            acc = rows_v[s, NB * t, pl.ds(16 * h, 16)]
                            for r in range(1, NB):
                                acc = acc + rows_v[s, NB * t + r,
                                                   pl.ds(16 * h, 16)]
                            out_v[b, t, pl.ds(16 * h, 16)] = acc
                    pltpu.async_copy(
                        out_v.at[b], out_hbm.at[pl.ds(base + CHUNK * c, CHUNK)],
                        ssems[b]
                    )

        wait_store(n_chunks - 2, 0)
        wait_store(n_chunks - 1, 1)

    return gather_sum


_PAD_B = 20480
_PAD_A = 10240
_R0_B, _R1_B = 1024, 256
_R0_A, _R1_A = 512, 128
_gather_sum_b = _make_gather_sum(_PAD_B, _R0_B, _R1_B)
_gather_sum_a = _make_gather_sum(_PAD_A, _R0_A, _R1_A)


def _prep_idx(g, n_pad):
    g = g.astype(jnp.int32)
    n = g.shape[0]
    g = jnp.concatenate([g, jnp.zeros((n_pad - n, NB), jnp.int32)], axis=0)
    return g.reshape(n_pad // CHUNK, 80)


_BB = 1000
_NTREE_BLK = NMESS // _BB
_NTAB_BLK = NTAB // _BB


def _binput_body(fb_ref, wi_ref, out_ref):
    out_ref[...] = jnp.dot(fb_ref[...], wi_ref[...],
                           preferred_element_type=jnp.float32)


def _tc_binput(fbonds, W_i):
    return pl.pallas_call(
        _binput_body,
        grid=(NBOND // _BB,),
        in_specs=[
            pl.BlockSpec((_BB, BF), lambda i: (i, 0)),
            pl.BlockSpec((BF, H), lambda i: (0, 0)),
        ],
        out_specs=pl.BlockSpec((_BB, H), lambda i: (i, 0)),
        out_shape=jax.ShapeDtypeStruct((NBOND, H), jnp.float32),
    )(fbonds, W_i)


def _table0_body(tree_ref, bin_ref, out_ref):
    i = pl.program_id(0)

    @pl.when(i < _NTREE_BLK)
    def _():
        out_ref[...] = tree_ref[...]

    @pl.when(i >= _NTREE_BLK)
    def _():
        out_ref[...] = jnp.maximum(bin_ref[...], 0.0)


def _tc_table0(tree, binput):
    return pl.pallas_call(
        _table0_body,
        grid=(_NTAB_BLK,),
        in_specs=[
            pl.BlockSpec((_BB, H), lambda i: (jnp.minimum(i, _NTREE_BLK - 1), 0)),
            pl.BlockSpec((_BB, H), lambda i: (jnp.maximum(i - _NTREE_BLK, 0), 0)),
        ],
        out_specs=pl.BlockSpec((_BB, H), lambda i: (i, 0)),
        out_shape=jax.ShapeDtypeStruct((NTAB, H), jnp.float32),
    )(tree, binput)


def _iter_body(tab_ref, bin_ref, nei_ref, wh_ref, out_ref):
    del tab_ref
    acc = jnp.dot(nei_ref[...], wh_ref[...], preferred_element_type=jnp.float32)
    out_ref[...] = jnp.maximum(bin_ref[...] + acc, 0.0)


def _tc_iter(table, binput, nei, W_h):
    return pl.pallas_call(
        _iter_body,
        grid=(NBOND // _BB,),
        in_specs=[
            pl.BlockSpec(memory_space=pl.ANY),
            pl.BlockSpec((_BB, H), lambda i: (i, 0)),
            pl.BlockSpec((_BB, H), lambda i: (i, 0)),
            pl.BlockSpec((H, H), lambda i: (0, 0)),
        ],
        out_specs=pl.BlockSpec((_BB, H), lambda i: (i + _NTREE_BLK, 0)),
        out_shape=jax.ShapeDtypeStruct((NTAB, H), jnp.float32),
        input_output_aliases={0: 0},
    )(table, binput, nei, W_h)


_AB = 2000
_MPB = _AB // APM


def _out_body(fa_ref, nei_ref, woa_ref, won_ref, bo_ref, out_ref):
    hdn = jnp.dot(fa_ref[...], woa_ref[...], preferred_element_type=jnp.float32)
    hdn += jnp.dot(nei_ref[...], won_ref[...], preferred_element_type=jnp.float32)
    hdn = jnp.maximum(hdn + bo_ref[...], 0.0)
    rows = lax.broadcasted_iota(jnp.int32, (_MPB, _AB), 1) // APM
    mols = lax.broadcasted_iota(jnp.int32, (_MPB, _AB), 0)
    pool = jnp.where(rows == mols, 1.0 / APM, 0.0)
    out_ref[...] = jnp.dot(pool, hdn, preferred_element_type=jnp.float32)


def _tc_out(fatoms, nei_a, W_oa, W_on, b_o):
    return pl.pallas_call(
        _out_body,
        grid=(NA // _AB,),
        in_specs=[
            pl.BlockSpec((_AB, AF), lambda i: (i, 0)),
            pl.BlockSpec((_AB, H), lambda i: (i, 0)),
            pl.BlockSpec((AF, H), lambda i: (0, 0)),
            pl.BlockSpec((H, H), lambda i: (0, 0)),
            pl.BlockSpec((1, H), lambda i: (0, 0)),
        ],
        out_specs=pl.BlockSpec((_MPB, H), lambda i: (i, 0)),
        out_shape=jax.ShapeDtypeStruct((NMOL, H), jnp.float32),
    )(fatoms, nei_a, W_oa, W_on, b_o)


def kernel(fatoms, fbonds, tree_message, W_i, W_h, W_o, b_o, agraph, bgraph):
    idx_b = _prep_idx(bgraph, _PAD_B)
    idx_a = _prep_idx(agraph, _PAD_A)

    binput = _tc_binput(fbonds, W_i)
    table = _tc_table0(tree_message, binput)
    for _ in range(3):
        nei = _gather_sum_b(table, idx_b)[:NBOND]
        table = _tc_iter(table, binput, nei, W_h)
    nei_a = _gather_sum_a(table, idx_a)[:NA]
    return _tc_out(fatoms, nei_a, W_o[:AF], W_o[AF:], b_o.reshape(1, H))
